# baseline (speedup 1.0000x reference)
import functools

import jax
import jax.numpy as jnp
from jax import lax
from jax.experimental import pallas as pl
from jax.experimental.pallas import tpu as pltpu
from jax.experimental.pallas import tpu_sc as plsc

_NUM_BINS = 15
_MIN = -3.0
_MAX = 3.0
_D = 64
_B = 16384
_NC = 2
_NS = 16
_NW = _NC * _NS
_BPW = _B // _NW
_L = 16

_SCALE = (_NUM_BINS - 1) / (_MAX - _MIN)
_MAGIC = 2.0 ** 23


def _sc_embed_t(r, table):
    mesh = plsc.VectorSubcoreMesh(core_axis_name="c", subcore_axis_name="s")

    @functools.partial(
        pl.kernel,
        mesh=mesh,
        out_type=jax.ShapeDtypeStruct((_D, _B), jnp.float32),
        compiler_params=pltpu.CompilerParams(
            use_tc_tiling_on_sc=True, needs_layout_passes=False),
        scratch_types=[
            pltpu.VMEM((_BPW,), jnp.float32),
            pltpu.VMEM((_NUM_BINS, _D), jnp.float32),
            pltpu.VMEM((_D, _BPW), jnp.float32),
        ],
    )
    def k(r_hbm, table_hbm, out_hbm, r_v, table_v, outt_v):
        wid = lax.axis_index("s") * _NC + lax.axis_index("c")
        base = wid * _BPW
        pltpu.sync_copy(table_hbm, table_v)
        pltpu.sync_copy(r_hbm.at[pl.ds(base, _BPW)], r_v)
        for i in range(_BPW // _L):
            rv = r_v[pl.ds(i * _L, _L)]
            t = jnp.minimum(jnp.maximum(rv, _MIN), _MAX)
            x = (t - _MIN) * jnp.float32(_SCALE)
            f = (x + _MAGIC) - _MAGIC
            idx = f.astype(jnp.int32)
            for d in range(_D):
                col = plsc.load_gather(
                    table_v, [idx, jnp.full((_L,), d, jnp.int32)])
                outt_v[d, pl.ds(i * _L, _L)] = col
        pltpu.sync_copy(outt_v, out_hbm.at[:, pl.ds(base, _BPW)])

    return k(r, table)


def kernel(r, table):
    return _sc_embed_t(r, table).T


# speedup vs baseline: 2.2715x; 2.2715x over previous
import functools

import jax
import jax.numpy as jnp
from jax import lax
from jax.experimental import pallas as pl
from jax.experimental.pallas import tpu as pltpu
from jax.experimental.pallas import tpu_sc as plsc

_NUM_BINS = 15
_MIN = -3.0
_MAX = 3.0
_D = 64
_B = 16384
_NC = 2
_NS = 16
_NW = _NC * _NS
_BPW = _B // _NW
_L = 16

_SCALE = (_NUM_BINS - 1) / (_MAX - _MIN)
_MAGIC = 2.0 ** 23


def _sc_embed_t(r, table):
    mesh = plsc.VectorSubcoreMesh(core_axis_name="c", subcore_axis_name="s")

    @functools.partial(
        pl.kernel,
        mesh=mesh,
        out_type=jax.ShapeDtypeStruct((_D, _B), jnp.float32),
        compiler_params=pltpu.CompilerParams(
            use_tc_tiling_on_sc=True, needs_layout_passes=False),
        scratch_types=[
            pltpu.VMEM((_BPW,), jnp.float32),
            pltpu.VMEM((_NUM_BINS, _D), jnp.float32),
            pltpu.VMEM((_NUM_BINS * (_D + 1) + 1,), jnp.float32),
            pltpu.VMEM((_D, _BPW), jnp.float32),
        ],
    )
    def k(r_hbm, table_hbm, out_hbm, r_v, table_v, t65_v, outt_v):
        wid = lax.axis_index("s") * _NC + lax.axis_index("c")
        base = wid * _BPW
        pltpu.sync_copy(table_hbm, table_v)
        pltpu.sync_copy(r_hbm.at[pl.ds(base, _BPW)], r_v)
        for b in range(_NUM_BINS):
            for c in range(_D // _L):
                t65_v[pl.ds(b * (_D + 1) + c * _L, _L)] = (
                    table_v[b, pl.ds(c * _L, _L)])
        @plsc.parallel_loop(0, _BPW // _L)
        def _group(i):
            rv = r_v[pl.ds(i * _L, _L)]
            t = jnp.minimum(jnp.maximum(rv, _MIN), _MAX)
            x = (t - _MIN) * jnp.float32(_SCALE)
            f = (x + _MAGIC) - _MAGIC
            idx = f.astype(jnp.int32)
            a65 = idx * (_D + 1)
            for d in range(_D):
                col = plsc.load_gather(t65_v, [a65 + d])
                outt_v[d, pl.ds(i * _L, _L)] = col
        pltpu.sync_copy(outt_v, out_hbm.at[:, pl.ds(base, _BPW)])

    return k(r, table)


def kernel(r, table):
    return _sc_embed_t(r, table).T
